# Initial kernel scaffold; baseline (speedup 1.0000x reference)
import jax
import jax.numpy as jnp
from jax.experimental import pallas as pl


def kernel(pc, sp, bp, ax, memory):
    raise NotImplementedError("write your pallas kernel here")


# speedup vs baseline: 4.5772x; 4.5772x over previous
import functools

import jax
import jax.numpy as jnp
from jax import lax
from jax.experimental import pallas as pl
from jax.experimental.pallas import tpu as pltpu
from jax.experimental.pallas import tpu_sc as plsc

M = 65536
B = 256
LANES = 16
N_WORKERS = B // LANES
BLK = 2048


def _sc_gather_result(pc, sp, memory):
    mesh = plsc.VectorSubcoreMesh(core_axis_name="c", subcore_axis_name="s")
    info = plsc.get_sparse_core_info()
    nc = info.num_cores

    @functools.partial(
        pl.kernel,
        mesh=mesh,
        out_type=jax.ShapeDtypeStruct((B,), jnp.float32),
        scratch_types=[
            pltpu.VMEM((LANES,), jnp.int32),
            pltpu.VMEM((LANES,), jnp.int32),
            pltpu.VMEM((LANES,), jnp.float32),
            pltpu.VMEM((LANES,), jnp.float32),
            pltpu.VMEM((LANES,), jnp.float32),
            pltpu.SemaphoreType.DMA,
        ],
    )
    def k(pc_hbm, sp_hbm, mem_hbm, out_hbm, pc_v, sp_v, instr_v, stk_v, res_v, sem):
        wid = lax.axis_index("s") * nc + lax.axis_index("c")

        @pl.when(wid < N_WORKERS)
        def _():
            base = wid * LANES
            pltpu.sync_copy(pc_hbm.at[pl.ds(base, LANES)], pc_v)
            pltpu.sync_copy(sp_hbm.at[pl.ds(base, LANES)], sp_v)
            pltpu.async_copy(mem_hbm.at[pc_v], instr_v, sem).wait()
            pltpu.async_copy(mem_hbm.at[sp_v], stk_v, sem).wait()
            instr = instr_v[...]
            y = instr * (1.0 / 256.0)
            t = y.astype(jnp.int32).astype(jnp.float32)
            imm = jnp.where(t > y, t - 1.0, t)
            res_v[...] = stk_v[...] + imm
            pltpu.sync_copy(res_v, out_hbm.at[pl.ds(base, LANES)])

    return k(pc, sp, memory)


def _tc_broadcast(memory2d, sp2d, result2d):
    def body(mem_ref, sp_ref, res_ref, out_ref):
        j = pl.program_id(0)
        cols = j * BLK + lax.broadcasted_iota(jnp.int32, (B, BLK), 1)
        out_ref[...] = jnp.where(cols == sp_ref[...], res_ref[...], mem_ref[...])

    return pl.pallas_call(
        body,
        grid=(M // BLK,),
        in_specs=[
            pl.BlockSpec((1, BLK), lambda j: (0, j)),
            pl.BlockSpec((B, 1), lambda j: (0, 0)),
            pl.BlockSpec((B, 1), lambda j: (0, 0)),
        ],
        out_specs=pl.BlockSpec((B, BLK), lambda j: (0, j)),
        out_shape=jax.ShapeDtypeStruct((B, M), jnp.float32),
    )(memory2d, sp2d, result2d)


def kernel(pc, sp, bp, ax, memory):
    pc = pc.astype(jnp.int32)
    sp = sp.astype(jnp.int32)
    result = _sc_gather_result(pc, sp, memory)
    return _tc_broadcast(
        memory.reshape(1, M), sp.reshape(B, 1), result.reshape(B, 1)
    )
